# baseline (speedup 1.0000x reference)
import functools

import jax
import jax.numpy as jnp
from jax import lax
from jax.experimental import pallas as pl
from jax.experimental.pallas import tpu as pltpu
from jax.experimental.pallas import tpu_sc as plsc

MEM_ROWS = 65536
BATCH = 4096
FEAT_DIM = 256
NUM_CORES = 2
NUM_SUBCORES = 16
NUM_WORKERS = NUM_CORES * NUM_SUBCORES

SPLIT = 40960
SC_ROWS = MEM_ROWS - SPLIT
ROWS_PER_W = SC_ROWS // NUM_WORKERS
CHUNK = 128
NCHUNK = ROWS_PER_W // CHUNK
NBUF = 3
AHEAD = 1

TC_BLOCK = 1024
XBLK = BATCH // TC_BLOCK


def _sc_copy(feats):
    mesh = plsc.VectorSubcoreMesh(
        core_axis_name="core", subcore_axis_name="subcore"
    )

    @functools.partial(
        pl.kernel,
        out_type=jax.ShapeDtypeStruct((SC_ROWS, FEAT_DIM), jnp.float32),
        mesh=mesh,
        scratch_types=[
            pltpu.VMEM((NBUF, CHUNK, FEAT_DIM), jnp.float32),
            pltpu.SemaphoreType.DMA((NBUF,)),
            pltpu.SemaphoreType.DMA((NBUF,)),
        ],
    )
    def bank(f_hbm, o_hbm, buf, gsem, ssem):
        wid = lax.axis_index("subcore") * NUM_CORES + lax.axis_index("core")
        base = wid * ROWS_PER_W
        gathers, scatters = [None] * NCHUNK, [None] * NCHUNK

        def fire_gather(i):
            b = i % NBUF
            if i >= NBUF:
                scatters[i - NBUF].wait()
            gathers[i] = pltpu.make_async_copy(
                f_hbm.at[pl.ds(SPLIT + base + i * CHUNK, CHUNK)],
                buf.at[b], gsem.at[b])
            gathers[i].start()

        for i in range(AHEAD):
            fire_gather(i)
        for i in range(NCHUNK):
            if i + AHEAD < NCHUNK:
                fire_gather(i + AHEAD)
            b = i % NBUF
            gathers[i].wait()
            scatters[i] = pltpu.make_async_copy(
                buf.at[b],
                o_hbm.at[pl.ds(base + i * CHUNK, CHUNK)], ssem.at[b])
            scatters[i].start()
        for i in range(NCHUNK - NBUF, NCHUNK):
            scatters[i].wait()

    return bank(feats)


def _tc_copy_body(x_ref, f_ref, o_ref):
    i = pl.program_id(0)

    @pl.when(i < XBLK)
    def _():
        o_ref[...] = x_ref[...]

    @pl.when(i >= XBLK)
    def _():
        o_ref[...] = f_ref[...]


def _tc_copy(x, feats):
    return pl.pallas_call(
        _tc_copy_body,
        grid=(SPLIT // TC_BLOCK,),
        in_specs=[
            pl.BlockSpec((TC_BLOCK, FEAT_DIM),
                         lambda i: (jnp.minimum(i, XBLK - 1), 0)),
            pl.BlockSpec((TC_BLOCK, FEAT_DIM),
                         lambda i: (jnp.maximum(i, XBLK), 0)),
        ],
        out_specs=pl.BlockSpec((TC_BLOCK, FEAT_DIM), lambda i: (i, 0)),
        out_shape=jax.ShapeDtypeStruct((SPLIT, FEAT_DIM), jnp.float32),
    )(x, feats)


def kernel(x, feats):
    top = _tc_copy(x, feats)
    bot = _sc_copy(feats)
    return jnp.concatenate([top, bot], axis=0)


# speedup vs baseline: 1.3156x; 1.3156x over previous
import functools

import jax
import jax.numpy as jnp
from jax import lax
from jax.experimental import pallas as pl
from jax.experimental.pallas import tpu as pltpu
from jax.experimental.pallas import tpu_sc as plsc

MEM_ROWS = 65536
BATCH = 4096
FEAT_DIM = 256
NUM_CORES = 2
NUM_SUBCORES = 16
NUM_WORKERS = NUM_CORES * NUM_SUBCORES

SPLIT = 49152
SC_ROWS = MEM_ROWS - SPLIT
ROWS_PER_W = SC_ROWS // NUM_WORKERS
CHUNK = 128
NCHUNK = ROWS_PER_W // CHUNK
NBUF = 3
AHEAD = 1

TC_BLOCK = 1024
XBLK = BATCH // TC_BLOCK


def _sc_copy(feats):
    mesh = plsc.VectorSubcoreMesh(
        core_axis_name="core", subcore_axis_name="subcore"
    )

    @functools.partial(
        pl.kernel,
        out_type=jax.ShapeDtypeStruct((SC_ROWS, FEAT_DIM), jnp.float32),
        mesh=mesh,
        scratch_types=[
            pltpu.VMEM((NBUF, CHUNK, FEAT_DIM), jnp.float32),
            pltpu.SemaphoreType.DMA((NBUF,)),
            pltpu.SemaphoreType.DMA((NBUF,)),
        ],
    )
    def bank(f_hbm, o_hbm, buf, gsem, ssem):
        wid = lax.axis_index("subcore") * NUM_CORES + lax.axis_index("core")
        base = wid * ROWS_PER_W
        gathers, scatters = [None] * NCHUNK, [None] * NCHUNK

        def fire_gather(i):
            b = i % NBUF
            if i >= NBUF:
                scatters[i - NBUF].wait()
            gathers[i] = pltpu.make_async_copy(
                f_hbm.at[pl.ds(SPLIT + base + i * CHUNK, CHUNK)],
                buf.at[b], gsem.at[b])
            gathers[i].start()

        for i in range(AHEAD):
            fire_gather(i)
        for i in range(NCHUNK):
            if i + AHEAD < NCHUNK:
                fire_gather(i + AHEAD)
            b = i % NBUF
            gathers[i].wait()
            scatters[i] = pltpu.make_async_copy(
                buf.at[b],
                o_hbm.at[pl.ds(base + i * CHUNK, CHUNK)], ssem.at[b])
            scatters[i].start()
        for i in range(NCHUNK - NBUF, NCHUNK):
            scatters[i].wait()

    return bank(feats)


def _tc_copy_body(x_ref, f_ref, o_ref):
    i = pl.program_id(0)

    @pl.when(i < XBLK)
    def _():
        o_ref[...] = x_ref[...]

    @pl.when(i >= XBLK)
    def _():
        o_ref[...] = f_ref[...]


def _tc_copy(x, feats):
    return pl.pallas_call(
        _tc_copy_body,
        grid=(SPLIT // TC_BLOCK,),
        in_specs=[
            pl.BlockSpec((TC_BLOCK, FEAT_DIM),
                         lambda i: (jnp.minimum(i, XBLK - 1), 0)),
            pl.BlockSpec((TC_BLOCK, FEAT_DIM),
                         lambda i: (jnp.maximum(i, XBLK), 0)),
        ],
        out_specs=pl.BlockSpec((TC_BLOCK, FEAT_DIM), lambda i: (i, 0)),
        out_shape=jax.ShapeDtypeStruct((MEM_ROWS, FEAT_DIM), jnp.float32),
    )(x, feats)


def kernel(x, feats):
    big = _tc_copy(x, feats)
    bot = _sc_copy(feats)
    return jax.lax.dynamic_update_slice(big, bot, (SPLIT, 0))


# speedup vs baseline: 1.6705x; 1.2697x over previous
import functools

import jax
import jax.numpy as jnp
from jax import lax
from jax.experimental import pallas as pl
from jax.experimental.pallas import tpu as pltpu
from jax.experimental.pallas import tpu_sc as plsc

MEM_ROWS = 65536
BATCH = 4096
FEAT_DIM = 256
NUM_CORES = 2
NUM_SUBCORES = 16
NUM_WORKERS = NUM_CORES * NUM_SUBCORES

SPLIT = 65536
SC_ROWS = MEM_ROWS - SPLIT
ROWS_PER_W = SC_ROWS // NUM_WORKERS
CHUNK = 128
NCHUNK = ROWS_PER_W // CHUNK
NBUF = 3
AHEAD = 1

TC_BLOCK = 1024
XBLK = BATCH // TC_BLOCK


def _sc_copy(feats):
    mesh = plsc.VectorSubcoreMesh(
        core_axis_name="core", subcore_axis_name="subcore"
    )

    @functools.partial(
        pl.kernel,
        out_type=jax.ShapeDtypeStruct((SC_ROWS, FEAT_DIM), jnp.float32),
        mesh=mesh,
        scratch_types=[
            pltpu.VMEM((NBUF, CHUNK, FEAT_DIM), jnp.float32),
            pltpu.SemaphoreType.DMA((NBUF,)),
            pltpu.SemaphoreType.DMA((NBUF,)),
        ],
    )
    def bank(f_hbm, o_hbm, buf, gsem, ssem):
        wid = lax.axis_index("subcore") * NUM_CORES + lax.axis_index("core")
        base = wid * ROWS_PER_W
        gathers, scatters = [None] * NCHUNK, [None] * NCHUNK

        def fire_gather(i):
            b = i % NBUF
            if i >= NBUF:
                scatters[i - NBUF].wait()
            gathers[i] = pltpu.make_async_copy(
                f_hbm.at[pl.ds(SPLIT + base + i * CHUNK, CHUNK)],
                buf.at[b], gsem.at[b])
            gathers[i].start()

        for i in range(AHEAD):
            fire_gather(i)
        for i in range(NCHUNK):
            if i + AHEAD < NCHUNK:
                fire_gather(i + AHEAD)
            b = i % NBUF
            gathers[i].wait()
            scatters[i] = pltpu.make_async_copy(
                buf.at[b],
                o_hbm.at[pl.ds(base + i * CHUNK, CHUNK)], ssem.at[b])
            scatters[i].start()
        for i in range(NCHUNK - NBUF, NCHUNK):
            scatters[i].wait()

    return bank(feats)


def _tc_copy_body(x_ref, f_ref, o_ref):
    i = pl.program_id(0)

    @pl.when(i < XBLK)
    def _():
        o_ref[...] = x_ref[...]

    @pl.when(i >= XBLK)
    def _():
        o_ref[...] = f_ref[...]


def _tc_copy(x, feats):
    return pl.pallas_call(
        _tc_copy_body,
        grid=(SPLIT // TC_BLOCK,),
        in_specs=[
            pl.BlockSpec((TC_BLOCK, FEAT_DIM),
                         lambda i: (jnp.minimum(i, XBLK - 1), 0)),
            pl.BlockSpec((TC_BLOCK, FEAT_DIM),
                         lambda i: (jnp.maximum(i, XBLK), 0)),
        ],
        out_specs=pl.BlockSpec((TC_BLOCK, FEAT_DIM), lambda i: (i, 0)),
        out_shape=jax.ShapeDtypeStruct((MEM_ROWS, FEAT_DIM), jnp.float32),
    )(x, feats)


def kernel(x, feats):
    return _tc_copy(x, feats)
